# baseline (speedup 1.0000x reference)
import functools

import jax
import jax.numpy as jnp
from jax import lax
from jax.experimental import pallas as pl
from jax.experimental.pallas import tpu as pltpu
from jax.experimental.pallas import tpu_sc as plsc

_B = 16384
_D = 32
_CHUNK = 128


@functools.cache
def _build(num_cores, num_subcores, b_per_w):
    n_chunks = b_per_w // _CHUNK
    mesh = plsc.VectorSubcoreMesh(core_axis_name="c", subcore_axis_name="s")

    @functools.partial(
        pl.kernel,
        mesh=mesh,
        out_type=jax.ShapeDtypeStruct((_B, _D), jnp.float32),
        scratch_types=[
            pltpu.VMEM((n_chunks, _CHUNK), jnp.int32),
            pltpu.VMEM((b_per_w, _D), jnp.float32),
            pltpu.SemaphoreType.DMA,
        ],
        compiler_params=pltpu.CompilerParams(use_tc_tiling_on_sc=False),
    )
    def k(idx_hbm, table_hbm, out_hbm, idx_v, rows_v, sem):
        wid = lax.axis_index("s") * num_cores + lax.axis_index("c")
        base = wid * b_per_w
        pltpu.sync_copy(idx_hbm.at[pl.ds(wid * n_chunks, n_chunks)], idx_v)
        copies = []
        for j in range(n_chunks):
            copies.append(
                pltpu.async_copy(
                    table_hbm.at[idx_v.at[j]],
                    rows_v.at[pl.ds(j * _CHUNK, _CHUNK)],
                    sem,
                )
            )
        for c in copies:
            c.wait()
        pltpu.sync_copy(rows_v, out_hbm.at[pl.ds(base, b_per_w)])

    return k


def kernel(index, table):
    info = plsc.get_sparse_core_info()
    nw = info.num_cores * info.num_subcores
    b_per_w = _B // nw
    idx2d = index.reshape(_B // _CHUNK, _CHUNK)
    return _build(info.num_cores, info.num_subcores, b_per_w)(idx2d, table)


# speedup vs baseline: 3.6157x; 3.6157x over previous
import functools

import jax
import jax.numpy as jnp
from jax import lax
from jax.experimental import pallas as pl
from jax.experimental.pallas import tpu as pltpu
from jax.experimental.pallas import tpu_sc as plsc

_B = 16384
_D = 32
_CHUNK = 16


@functools.cache
def _build(num_cores, num_subcores):
    nw = num_cores * num_subcores
    b_per_w = _B // nw
    n_chunks = b_per_w // _CHUNK
    mesh = plsc.VectorSubcoreMesh(core_axis_name="c", subcore_axis_name="s")

    @functools.partial(
        pl.kernel,
        mesh=mesh,
        out_type=jax.ShapeDtypeStruct((_B * _D,), jnp.float32),
        scratch_types=[
            pltpu.VMEM((b_per_w,), jnp.int32),
            pltpu.VMEM((_CHUNK, 4, 8, 128), jnp.float32),
            pltpu.VMEM((b_per_w * _D,), jnp.float32),
            pltpu.SemaphoreType.DMA,
        ],
        compiler_params=pltpu.CompilerParams(
            use_tc_tiling_on_sc=True, needs_layout_passes=False
        ),
    )
    def k(idx_hbm, t2v_hbm, out_hbm, idx_v, stripes_v, rows_v, sem):
        wid = lax.axis_index("s") * num_cores + lax.axis_index("c")
        base = wid * b_per_w
        pltpu.sync_copy(idx_hbm.at[pl.ds(base, b_per_w)], idx_v)

        g_hi = lax.div(lax.iota(jnp.int32, 16), 8)
        s_idx = lax.rem(lax.iota(jnp.int32, 16), 8)

        def chunk_body(c, _):
            v = idx_v[pl.ds(c * _CHUNK, _CHUNK)]
            copies = []
            for j in range(_CHUNK):
                r = v[j]
                r128 = pl.multiple_of(
                    lax.shift_left(lax.shift_right_logical(r, 7), 7), 128
                )
                copies.append(
                    pltpu.async_copy(
                        t2v_hbm.at[:, :, pl.ds(r128, 128)],
                        stripes_v.at[j],
                        sem,
                    )
                )
            lanes = lax.rem(v, 128)
            for j in range(_CHUNK):
                copies[j].wait()
                l16 = jnp.full((16,), lanes[j], jnp.int32)
                lo = plsc.load_gather(stripes_v.at[j], [g_hi, s_idx, l16])
                hi = plsc.load_gather(
                    stripes_v.at[j], [g_hi + 2, s_idx, l16]
                )
                o = (c * _CHUNK + j) * _D
                rows_v[pl.ds(o, 16)] = lo
                rows_v[pl.ds(o + 16, 16)] = hi
            return ()

        lax.fori_loop(0, n_chunks, chunk_body, ())
        pltpu.sync_copy(rows_v, out_hbm.at[pl.ds(base * _D, b_per_w * _D)])

    return k


def kernel(index, table):
    info = plsc.get_sparse_core_info()
    t2v = table.T.reshape(4, 8, table.shape[0])
    out1d = _build(info.num_cores, info.num_subcores)(index, t2v)
    return out1d.reshape(_B, _D)
